# baseline (speedup 1.0000x reference)
import functools

import jax
import jax.numpy as jnp
from jax import lax
from jax.experimental import pallas as pl
from jax.experimental.pallas import tpu as pltpu
from jax.experimental.pallas import tpu_sc as plsc

N = 10000
NP = 10240
D = 128
E = 160000
ET = E + N
NC, NS, L = 2, 16, 16
NW = NC * NS
BLKA = 128
BLKB = 32
NRING = 3
ET_PAD = -(-ET // 6144) * 6144
CHUNK_A = ET_PAD // NS
CHUNK_B = ET_PAD // NW
NBLK_A = CHUNK_A // BLKA
NBLK_B = CHUNK_B // BLKB
ROWS_PER_TILE = NP // NS


def _mm_body(x_ref, w_ref, as_ref, ad_ref, h_ref, sd_ref):
    h = jnp.dot(x_ref[...], w_ref[...], preferred_element_type=jnp.float32)
    h_ref[...] = h
    s = jnp.sum(h * as_ref[...], axis=1)
    d = jnp.sum(h * ad_ref[...], axis=1)
    sd_ref[...] = jnp.stack([s, d], axis=0)


def _matmul(x, W, att_src2, att_dst2):
    return pl.pallas_call(
        _mm_body,
        out_shape=[
            jax.ShapeDtypeStruct((NP, D), jnp.float32),
            jax.ShapeDtypeStruct((2, NP), jnp.float32),
        ],
    )(x, W, att_src2, att_dst2)


_MESH = plsc.VectorSubcoreMesh(core_axis_name="c", subcore_axis_name="s")


@functools.partial(
    pl.kernel,
    out_type=jax.ShapeDtypeStruct((NC, NP, D), jnp.float32),
    mesh=_MESH,
    compiler_params=pltpu.CompilerParams(needs_layout_passes=False),
    scratch_types=[
        pltpu.VMEM((NP,), jnp.float32),
        pltpu.VMEM((NP,), jnp.float32),
        pltpu.VMEM((NP,), jnp.float32),
        [pltpu.VMEM((BLKA,), jnp.int32)] * 2,
        [pltpu.VMEM((BLKA,), jnp.int32)] * 2,
        [pltpu.VMEM((BLKA,), jnp.float32)] * 2,
        [pltpu.VMEM((BLKB,), jnp.int32)] * NRING,
        [pltpu.VMEM((BLKB,), jnp.int32)] * NRING,
        [pltpu.VMEM((BLKB,), jnp.float32)] * NRING,
        [pltpu.VMEM((BLKB,), jnp.int32)] * NRING,
        [pltpu.VMEM((BLKB, D), jnp.float32)] * NRING,
        pltpu.VMEM((BLKB,), jnp.float32),
        pltpu.VMEM_SHARED((NP,), jnp.float32),
        pltpu.VMEM_SHARED((NP, D), jnp.float32),
        [pltpu.SemaphoreType.DMA] * 2,
        [pltpu.SemaphoreType.DMA] * 2,
        [pltpu.SemaphoreType.DMA] * NRING,
        [pltpu.SemaphoreType.DMA] * NRING,
        [pltpu.SemaphoreType.DMA] * NRING,
    ],
)
def _sc_attention(h_hbm, sd_hbm, src_hbm, dst_hbm, ew_hbm, zro_hbm, zrov_hbm,
                  out_hbm,
                  s_loc, d_loc, den_loc, isrcA, idstA, exA,
                  isrcB, idstB, ewbB, dstblkB, rows, coef,
                  den_sh, out_sh, sem_i, sem_a, sem_ib, sem_g, sem_s):
    cid = lax.axis_index("c")
    sid = lax.axis_index("s")
    wid = sid * NC + cid
    r0 = sid * ROWS_PER_TILE
    a0 = sid * CHUNK_A
    b0 = wid * CHUNK_B
    lanes = lax.broadcasted_iota(jnp.int32, (L,), 0)

    init_scope = jax.named_scope("sc_init")
    init_scope.__enter__()
    pltpu.sync_copy(sd_hbm.at[0], s_loc)
    pltpu.sync_copy(sd_hbm.at[1], d_loc)
    pltpu.sync_copy(zro_hbm.at[pl.ds(r0, ROWS_PER_TILE)],
                    out_sh.at[pl.ds(r0, ROWS_PER_TILE)])
    pltpu.sync_copy(zrov_hbm.at[pl.ds(r0, ROWS_PER_TILE)],
                    den_sh.at[pl.ds(r0, ROWS_PER_TILE)])
    init_scope.__exit__(None, None, None)
    pa_scope = jax.named_scope("sc_phase_a")
    pa_scope.__enter__()
    plsc.subcore_barrier()

    def a_fetch_start(b, j):
        pltpu.async_copy(src_hbm.at[pl.ds(a0 + b * BLKA, BLKA)], isrcA[j],
                         sem_i[j])
        pltpu.async_copy(dst_hbm.at[pl.ds(a0 + b * BLKA, BLKA)], idstA[j],
                         sem_i[j])

    def a_fetch_wait(b, j):
        pltpu.make_async_copy(src_hbm.at[pl.ds(a0 + b * BLKA, BLKA)],
                              isrcA[j], sem_i[j]).wait()
        pltpu.make_async_copy(dst_hbm.at[pl.ds(a0 + b * BLKA, BLKA)],
                              idstA[j], sem_i[j]).wait()

    def a_scatter_wait(j):
        pltpu.make_async_copy(exA[j], den_sh.at[idstA[j]], sem_a[j]).wait()

    def a_compute(b, j):
        for g in range(BLKA // L):
            off = g * L
            iv = isrcA[j][pl.ds(off, L)]
            jv = idstA[j][pl.ds(off, L)]
            e = plsc.load_gather(s_loc, [iv]) + plsc.load_gather(d_loc, [jv])
            e = jnp.where(e >= 0.0, e, 0.2 * e)
            ex = jnp.exp(e)
            ex = jnp.where(a0 + b * BLKA + off + lanes < ET, ex, 0.0)
            exA[j][pl.ds(off, L)] = ex

    a_fetch_start(0, 0)

    def a_pair(pi, carry):
        for j in range(2):
            b = 2 * pi + j
            j1 = 1 - j
            a_fetch_wait(b, j)

            @pl.when(b >= 1)
            def _():
                a_scatter_wait(j1)

            @pl.when(b + 1 < NBLK_A)
            def _():
                a_fetch_start(b + 1, j1)

            a_compute(b, j)
            pltpu.async_copy(exA[j], den_sh.at[idstA[j]], sem_a[j], add=True)
        return carry

    lax.fori_loop(0, NBLK_A // 2, a_pair, 0)
    a_scatter_wait((NBLK_A - 1) % 2)
    plsc.subcore_barrier()
    pltpu.sync_copy(den_sh, den_loc)
    pa_scope.__exit__(None, None, None)
    pb_scope = jax.named_scope("sc_phase_b")
    pb_scope.__enter__()

    def b_fetch_start(b, r):
        pltpu.async_copy(src_hbm.at[pl.ds(b0 + b * BLKB, BLKB)], isrcB[r],
                         sem_ib[r])
        pltpu.async_copy(dst_hbm.at[pl.ds(b0 + b * BLKB, BLKB)], idstB[r],
                         sem_ib[r])
        pltpu.async_copy(ew_hbm.at[pl.ds(b0 + b * BLKB, BLKB)], ewbB[r],
                         sem_ib[r])

    def b_fetch_wait(b, r):
        pltpu.make_async_copy(src_hbm.at[pl.ds(b0 + b * BLKB, BLKB)],
                              isrcB[r], sem_ib[r]).wait()
        pltpu.make_async_copy(dst_hbm.at[pl.ds(b0 + b * BLKB, BLKB)],
                              idstB[r], sem_ib[r]).wait()
        pltpu.make_async_copy(ew_hbm.at[pl.ds(b0 + b * BLKB, BLKB)],
                              ewbB[r], sem_ib[r]).wait()

    def b_gather_start(r):
        pltpu.async_copy(h_hbm.at[isrcB[r]], rows[r], sem_g[r])

    def b_gather_wait(r):
        pltpu.make_async_copy(h_hbm.at[isrcB[r]], rows[r], sem_g[r]).wait()

    def b_scatter_wait(r):
        pltpu.make_async_copy(rows[r], out_sh.at[dstblkB[r]], sem_s[r]).wait()

    def b_coef(b, r):
        for g in range(BLKB // L):
            off = g * L
            iv = isrcB[r][pl.ds(off, L)]
            jv = idstB[r][pl.ds(off, L)]
            dstblkB[r][pl.ds(off, L)] = jv
            e = plsc.load_gather(s_loc, [iv]) + plsc.load_gather(d_loc, [jv])
            e = jnp.where(e >= 0.0, e, 0.2 * e)
            ex = jnp.exp(e) * ewbB[r][pl.ds(off, L)]
            den = plsc.load_gather(den_loc, [jv])
            coef[pl.ds(off, L)] = ex / den

    def b_scale(r):
        def kloop(c, carry2):
            for ke in range(16):
                kf = jnp.zeros((L,), jnp.int32) + (c * 16 + ke)
                cvec = plsc.load_gather(coef, [kf])
                for g in range(D // L):
                    cols = lanes + g * L
                    v = plsc.load_gather(rows[r], [kf, cols])
                    plsc.store_scatter(rows[r], [kf, cols], v * cvec)
            return carry2

        lax.fori_loop(0, BLKB // 16, kloop, 0)

    for r in range(NRING):
        b_fetch_start(r, r)
    b_fetch_wait(0, 0)
    b_fetch_wait(1, 1)
    b_gather_start(0)

    def b_triple(ti, carry):
        for r in range(NRING):
            b = NRING * ti + r
            r1 = (r + 1) % NRING
            r2 = (r + 2) % NRING
            b_coef(b, r)

            @pl.when(b + 1 < NBLK_B)
            def _():
                @pl.when(b >= 2)
                def _():
                    b_scatter_wait(r1)

                b_gather_start(r1)

            b_gather_wait(r)
            b_scale(r)
            pltpu.async_copy(rows[r], out_sh.at[dstblkB[r]], sem_s[r],
                             add=True)

            @pl.when(b + 2 < NBLK_B)
            def _():
                b_fetch_wait(b + 2, r2)

                @pl.when(b + 3 < NBLK_B)
                def _():
                    b_fetch_start(b + 3, r)
        return carry

    lax.fori_loop(0, NBLK_B // NRING, b_triple, 0)
    for r in range(NRING):
        b_scatter_wait(r)
    pb_scope.__exit__(None, None, None)
    with jax.named_scope("sc_writeout"):
        plsc.subcore_barrier()
        pltpu.sync_copy(out_sh.at[pl.ds(r0, ROWS_PER_TILE)],
                        out_hbm.at[cid, pl.ds(r0, ROWS_PER_TILE)])


def _comb_body(p_ref, b_ref, o_ref):
    o_ref[...] = jnp.maximum(p_ref[0] + p_ref[1] + b_ref[...], 0.0)


def _combine(partial, bias2):
    BN = 2000
    return pl.pallas_call(
        _comb_body,
        grid=(N // BN,),
        in_specs=[
            pl.BlockSpec((NC, BN, D), lambda i: (0, i, 0)),
            pl.BlockSpec((1, D), lambda i: (0, 0)),
        ],
        out_specs=pl.BlockSpec((BN, D), lambda i: (i, 0)),
        out_shape=jax.ShapeDtypeStruct((N, D), jnp.float32),
    )(partial, bias2)


def kernel(input_x, edge_index, edge_weight, W, att_src, att_dst, bias):
    loop = jnp.arange(N, dtype=jnp.int32)
    pad = ET_PAD - ET
    zi = jnp.zeros((pad,), jnp.int32)
    zf = jnp.zeros((pad,), jnp.float32)
    src = jnp.concatenate([edge_index[0].astype(jnp.int32), loop, zi])
    dst = jnp.concatenate([edge_index[1].astype(jnp.int32), loop, zi])
    ew = jnp.concatenate([edge_weight.astype(jnp.float32),
                          jnp.ones((N,), jnp.float32), zf])
    zro = jnp.zeros((NP, D), jnp.float32)
    zrov = jnp.zeros((NP,), jnp.float32)

    xp = jnp.concatenate([input_x, jnp.zeros((NP - N, D), jnp.float32)])
    h, sd = _matmul(xp, W, att_src.reshape(1, D), att_dst.reshape(1, D))
    partial = _sc_attention(h, sd, src, dst, ew, zro, zrov)
    out = _combine(partial, bias.reshape(1, D))
    return out[None]


# speedup vs baseline: 1.1473x; 1.1473x over previous
import functools

import jax
import jax.numpy as jnp
from jax import lax
from jax.experimental import pallas as pl
from jax.experimental.pallas import tpu as pltpu
from jax.experimental.pallas import tpu_sc as plsc

N = 10000
NP = 10240
D = 128
E = 160000
ET = E + N
NC, NS, L = 2, 16, 16
NW = NC * NS
BLKA = 256
BLKB = 32
NRING = 3
ET_PAD = -(-ET // 12288) * 12288
CHUNK_A = ET_PAD // NS
CHUNK_B = ET_PAD // NW
NBLK_A = CHUNK_A // BLKA
NBLK_B = CHUNK_B // BLKB
ROWS_PER_TILE = NP // NS


def _mm_body(x_ref, w_ref, as_ref, ad_ref, h_ref, sd_ref):
    h = jnp.dot(x_ref[...], w_ref[...], preferred_element_type=jnp.float32)
    h_ref[...] = h
    s = jnp.sum(h * as_ref[...], axis=1)
    d = jnp.sum(h * ad_ref[...], axis=1)
    sd_ref[...] = jnp.stack([s, d], axis=0)


def _matmul(x, W, att_src2, att_dst2):
    return pl.pallas_call(
        _mm_body,
        out_shape=[
            jax.ShapeDtypeStruct((NP, D), jnp.float32),
            jax.ShapeDtypeStruct((2, NP), jnp.float32),
        ],
    )(x, W, att_src2, att_dst2)


_MESH = plsc.VectorSubcoreMesh(core_axis_name="c", subcore_axis_name="s")


@functools.partial(
    pl.kernel,
    out_type=jax.ShapeDtypeStruct((NC, NP, D), jnp.float32),
    mesh=_MESH,
    compiler_params=pltpu.CompilerParams(needs_layout_passes=False),
    scratch_types=[
        pltpu.VMEM((NP,), jnp.float32),
        pltpu.VMEM((NP,), jnp.float32),
        pltpu.VMEM((NP,), jnp.float32),
        [pltpu.VMEM((BLKA,), jnp.int32)] * 2,
        [pltpu.VMEM((BLKA,), jnp.int32)] * 2,
        [pltpu.VMEM((BLKA,), jnp.float32)] * 2,
        [pltpu.VMEM((BLKB,), jnp.int32)] * NRING,
        [pltpu.VMEM((BLKB,), jnp.int32)] * NRING,
        [pltpu.VMEM((BLKB,), jnp.float32)] * NRING,
        [pltpu.VMEM((BLKB,), jnp.int32)] * NRING,
        [pltpu.VMEM((BLKB, D), jnp.float32)] * NRING,
        pltpu.VMEM((BLKB,), jnp.float32),
        pltpu.VMEM_SHARED((NP,), jnp.float32),
        pltpu.VMEM_SHARED((NP, D), jnp.float32),
        [pltpu.SemaphoreType.DMA] * 2,
        [pltpu.SemaphoreType.DMA] * 2,
        [pltpu.SemaphoreType.DMA] * NRING,
        [pltpu.SemaphoreType.DMA] * NRING,
        [pltpu.SemaphoreType.DMA] * NRING,
    ],
)
def _sc_attention(h_hbm, sd_hbm, src_hbm, dst_hbm, ew_hbm, zro_hbm, zrov_hbm,
                  out_hbm,
                  s_loc, d_loc, den_loc, isrcA, idstA, exA,
                  isrcB, idstB, ewbB, dstblkB, rows, coef,
                  den_sh, out_sh, sem_i, sem_a, sem_ib, sem_g, sem_s):
    cid = lax.axis_index("c")
    sid = lax.axis_index("s")
    wid = sid * NC + cid
    r0 = sid * ROWS_PER_TILE
    a0 = sid * CHUNK_A
    b0 = wid * CHUNK_B
    lanes = lax.broadcasted_iota(jnp.int32, (L,), 0)

    init_scope = jax.named_scope("sc_init")
    init_scope.__enter__()
    pltpu.sync_copy(sd_hbm.at[0], s_loc)
    pltpu.sync_copy(sd_hbm.at[1], d_loc)
    pltpu.sync_copy(zro_hbm.at[pl.ds(r0, ROWS_PER_TILE)],
                    out_sh.at[pl.ds(r0, ROWS_PER_TILE)])
    pltpu.sync_copy(zrov_hbm.at[pl.ds(r0, ROWS_PER_TILE)],
                    den_sh.at[pl.ds(r0, ROWS_PER_TILE)])
    init_scope.__exit__(None, None, None)
    pa_scope = jax.named_scope("sc_phase_a")
    pa_scope.__enter__()
    plsc.subcore_barrier()

    def a_fetch_start(b, j):
        pltpu.async_copy(src_hbm.at[pl.ds(a0 + b * BLKA, BLKA)], isrcA[j],
                         sem_i[j])
        pltpu.async_copy(dst_hbm.at[pl.ds(a0 + b * BLKA, BLKA)], idstA[j],
                         sem_i[j])

    def a_fetch_wait(b, j):
        pltpu.make_async_copy(src_hbm.at[pl.ds(a0 + b * BLKA, BLKA)],
                              isrcA[j], sem_i[j]).wait()
        pltpu.make_async_copy(dst_hbm.at[pl.ds(a0 + b * BLKA, BLKA)],
                              idstA[j], sem_i[j]).wait()

    def a_scatter_wait(j):
        pltpu.make_async_copy(exA[j], den_sh.at[idstA[j]], sem_a[j]).wait()

    def a_compute(b, j):
        for g in range(BLKA // L):
            off = g * L
            iv = isrcA[j][pl.ds(off, L)]
            jv = idstA[j][pl.ds(off, L)]
            e = plsc.load_gather(s_loc, [iv]) + plsc.load_gather(d_loc, [jv])
            e = jnp.where(e >= 0.0, e, 0.2 * e)
            ex = jnp.exp(e)
            ex = jnp.where(a0 + b * BLKA + off + lanes < ET, ex, 0.0)
            exA[j][pl.ds(off, L)] = ex

    a_fetch_start(0, 0)

    def a_pair(pi, carry):
        for j in range(2):
            b = 2 * pi + j
            j1 = 1 - j
            a_fetch_wait(b, j)

            @pl.when(b >= 1)
            def _():
                a_scatter_wait(j1)

            @pl.when(b + 1 < NBLK_A)
            def _():
                a_fetch_start(b + 1, j1)

            a_compute(b, j)
            pltpu.async_copy(exA[j], den_sh.at[idstA[j]], sem_a[j], add=True)
        return carry

    lax.fori_loop(0, NBLK_A // 2, a_pair, 0)
    a_scatter_wait((NBLK_A - 1) % 2)
    plsc.subcore_barrier()
    pltpu.sync_copy(den_sh, den_loc)
    pa_scope.__exit__(None, None, None)
    pb_scope = jax.named_scope("sc_phase_b")
    pb_scope.__enter__()

    def b_fetch_start(b, r):
        pltpu.async_copy(src_hbm.at[pl.ds(b0 + b * BLKB, BLKB)], isrcB[r],
                         sem_ib[r])
        pltpu.async_copy(dst_hbm.at[pl.ds(b0 + b * BLKB, BLKB)], idstB[r],
                         sem_ib[r])
        pltpu.async_copy(ew_hbm.at[pl.ds(b0 + b * BLKB, BLKB)], ewbB[r],
                         sem_ib[r])

    def b_fetch_wait(b, r):
        pltpu.make_async_copy(src_hbm.at[pl.ds(b0 + b * BLKB, BLKB)],
                              isrcB[r], sem_ib[r]).wait()
        pltpu.make_async_copy(dst_hbm.at[pl.ds(b0 + b * BLKB, BLKB)],
                              idstB[r], sem_ib[r]).wait()
        pltpu.make_async_copy(ew_hbm.at[pl.ds(b0 + b * BLKB, BLKB)],
                              ewbB[r], sem_ib[r]).wait()

    def b_gather_start(r):
        pltpu.async_copy(h_hbm.at[isrcB[r]], rows[r], sem_g[r])

    def b_gather_wait(r):
        pltpu.make_async_copy(h_hbm.at[isrcB[r]], rows[r], sem_g[r]).wait()

    def b_scatter_wait(r):
        pltpu.make_async_copy(rows[r], out_sh.at[dstblkB[r]], sem_s[r]).wait()

    def b_coef(b, r):
        for g in range(BLKB // L):
            off = g * L
            iv = isrcB[r][pl.ds(off, L)]
            jv = idstB[r][pl.ds(off, L)]
            dstblkB[r][pl.ds(off, L)] = jv
            e = plsc.load_gather(s_loc, [iv]) + plsc.load_gather(d_loc, [jv])
            e = jnp.where(e >= 0.0, e, 0.2 * e)
            ex = jnp.exp(e) * ewbB[r][pl.ds(off, L)]
            den = plsc.load_gather(den_loc, [jv])
            coef[pl.ds(off, L)] = ex / den

    def b_scale(r):
        def kloop(c, carry2):
            for ke in range(8):
                kf = jnp.zeros((L,), jnp.int32) + (c * 8 + ke)
                cvec = plsc.load_gather(coef, [kf])
                for g in range(D // L):
                    cols = lanes + g * L
                    v = plsc.load_gather(rows[r], [kf, cols])
                    plsc.store_scatter(rows[r], [kf, cols], v * cvec)
            return carry2

        lax.fori_loop(0, BLKB // 8, kloop, 0)

    for r in range(NRING):
        b_fetch_start(r, r)
    b_fetch_wait(0, 0)
    b_fetch_wait(1, 1)
    b_gather_start(0)

    def b_triple(ti, carry):
        for r in range(NRING):
            b = NRING * ti + r
            r1 = (r + 1) % NRING
            r2 = (r + 2) % NRING
            b_coef(b, r)

            @pl.when(b + 1 < NBLK_B)
            def _():
                @pl.when(b >= 2)
                def _():
                    b_scatter_wait(r1)

                b_gather_start(r1)

            b_gather_wait(r)
            b_scale(r)
            pltpu.async_copy(rows[r], out_sh.at[dstblkB[r]], sem_s[r],
                             add=True)

            @pl.when(b + 2 < NBLK_B)
            def _():
                b_fetch_wait(b + 2, r2)

                @pl.when(b + 3 < NBLK_B)
                def _():
                    b_fetch_start(b + 3, r)
        return carry

    lax.fori_loop(0, NBLK_B // NRING, b_triple, 0)
    for r in range(NRING):
        b_scatter_wait(r)
    pb_scope.__exit__(None, None, None)
    with jax.named_scope("sc_writeout"):
        plsc.subcore_barrier()
        pltpu.sync_copy(out_sh.at[pl.ds(r0, ROWS_PER_TILE)],
                        out_hbm.at[cid, pl.ds(r0, ROWS_PER_TILE)])


def _comb_body(p_ref, b_ref, o_ref):
    o_ref[...] = jnp.maximum(p_ref[0] + p_ref[1] + b_ref[...], 0.0)


def _combine(partial, bias2):
    BN = 2000
    return pl.pallas_call(
        _comb_body,
        grid=(N // BN,),
        in_specs=[
            pl.BlockSpec((NC, BN, D), lambda i: (0, i, 0)),
            pl.BlockSpec((1, D), lambda i: (0, 0)),
        ],
        out_specs=pl.BlockSpec((BN, D), lambda i: (i, 0)),
        out_shape=jax.ShapeDtypeStruct((N, D), jnp.float32),
    )(partial, bias2)


def kernel(input_x, edge_index, edge_weight, W, att_src, att_dst, bias):
    loop = jnp.arange(N, dtype=jnp.int32)
    pad = ET_PAD - ET
    zi = jnp.zeros((pad,), jnp.int32)
    zf = jnp.zeros((pad,), jnp.float32)
    src = jnp.concatenate([edge_index[0].astype(jnp.int32), loop, zi])
    dst = jnp.concatenate([edge_index[1].astype(jnp.int32), loop, zi])
    ew = jnp.concatenate([edge_weight.astype(jnp.float32),
                          jnp.ones((N,), jnp.float32), zf])
    zro = jnp.zeros((NP, D), jnp.float32)
    zrov = jnp.zeros((NP,), jnp.float32)

    xp = jnp.concatenate([input_x, jnp.zeros((NP - N, D), jnp.float32)])
    h, sd = _matmul(xp, W, att_src.reshape(1, D), att_dst.reshape(1, D))
    partial = _sc_attention(h, sd, src, dst, ew, zro, zrov)
    out = _combine(partial, bias.reshape(1, D))
    return out[None]
